# baseline (speedup 1.0000x reference)
import functools
import math

import numpy as np
import jax
import jax.numpy as jnp
from jax.experimental import pallas as pl


FFT_TOPK = 8


def _dft_weights(n: int, fp: int) -> np.ndarray:
    f = n // 2 + 1
    kk = np.arange(f, dtype=np.float64)
    nn = np.arange(n, dtype=np.float64)
    ang = 2.0 * np.pi * np.outer(nn, kk) / n
    w = np.zeros((n, 2 * fp), dtype=np.float64)
    w[:, :f] = np.cos(ang)
    w[:, fp:fp + f] = np.sin(ang)
    return w.astype(np.float32)


def _fourier_loss_block(xo_ref, xt_ref, w_ref, out_ref, *, f, fp, n_valid):
    i = pl.program_id(0)

    w = w_ref[...]
    om = jnp.dot(xo_ref[...], w, preferred_element_type=jnp.float32)
    tm = jnp.dot(xt_ref[...], w, preferred_element_type=jnp.float32)
    del w

    o2 = om[:, :fp] ** 2 + om[:, fp:] ** 2
    t2 = tm[:, :fp] ** 2 + tm[:, fp:] ** 2

    r = o2.shape[0]
    iota = jax.lax.broadcasted_iota(jnp.int32, (r, fp), 1)
    valid = iota < f
    o2 = jnp.where(valid, o2, 0.0)
    t2 = jnp.where(valid, t2, -1.0)

    rowsum = jnp.sum(o2, axis=1)

    adj = jnp.zeros((r,), dtype=jnp.float32)
    for _ in range(FFT_TOPK):
        m = jnp.max(t2, axis=1, keepdims=True)
        cand = jnp.where(t2 == m, iota, fp)
        amin = jnp.min(cand, axis=1, keepdims=True)
        onehot = iota == amin
        osel = jnp.sum(jnp.where(onehot, o2, 0.0), axis=1)
        tsel = m[:, 0]
        adj = adj + tsel - 2.0 * jnp.sqrt(jnp.maximum(osel * tsel, 0.0))
        t2 = jnp.where(onehot, -1.0, t2)

    scale2 = 1.0 / float(n_valid)
    total = (rowsum + adj) * scale2
    rowloss = jnp.sqrt(jnp.maximum(total, 0.0))
    partial = jnp.sum(rowloss).reshape(1, 1)

    @pl.when(i == 0)
    def _init():
        out_ref[...] = jnp.zeros((1, 1), jnp.float32)

    out_ref[...] += partial


@functools.partial(jax.jit, static_argnames=("block_rows",))
def _fourier_loss(output, target, block_rows=256):
    b, n = output.shape
    f = n // 2 + 1
    fp = ((f + 127) // 128) * 128
    w = jnp.asarray(_dft_weights(n, fp), dtype=jnp.bfloat16)

    grid = (b // block_rows,)
    out = pl.pallas_call(
        functools.partial(_fourier_loss_block, f=f, fp=fp, n_valid=n),
        grid=grid,
        in_specs=[
            pl.BlockSpec((block_rows, n), lambda i: (i, 0)),
            pl.BlockSpec((block_rows, n), lambda i: (i, 0)),
            pl.BlockSpec((n, 2 * fp), lambda i: (0, 0)),
        ],
        out_specs=pl.BlockSpec((1, 1), lambda i: (0, 0)),
        out_shape=jax.ShapeDtypeStruct((1, 1), jnp.float32),
    )(output.astype(jnp.bfloat16), target.astype(jnp.bfloat16), w)
    return out[0, 0] / b


def kernel(output, target):
    return _fourier_loss(output, target)


# speedup vs baseline: 1.2240x; 1.2240x over previous
import functools
import math

import numpy as np
import jax
import jax.numpy as jnp
from jax.experimental import pallas as pl


FFT_TOPK = 8


def _dft_weights(n: int, fp: int) -> np.ndarray:
    f = n // 2 + 1
    kk = np.arange(f, dtype=np.float64)
    nn = np.arange(n, dtype=np.float64)
    ang = 2.0 * np.pi * np.outer(nn, kk) / n
    w = np.zeros((n, 2 * fp), dtype=np.float64)
    w[:, :f] = np.cos(ang)
    w[:, fp:fp + f] = np.sin(ang)
    return w.astype(np.float32)


def _fourier_loss_block(xo_ref, xt_ref, w_ref, out_ref, *, f, fp, n_valid):
    i = pl.program_id(0)

    w = w_ref[...]
    om = jnp.dot(xo_ref[...].astype(jnp.bfloat16), w,
                 preferred_element_type=jnp.float32)
    tm = jnp.dot(xt_ref[...].astype(jnp.bfloat16), w,
                 preferred_element_type=jnp.float32)
    del w

    o2 = om[:, :fp] ** 2 + om[:, fp:] ** 2
    t2 = tm[:, :fp] ** 2 + tm[:, fp:] ** 2

    r = o2.shape[0]
    iota = jax.lax.broadcasted_iota(jnp.int32, (r, fp), 1)
    valid = iota < f
    o2 = jnp.where(valid, o2, 0.0)
    t2 = jnp.where(valid, t2, -1.0)

    rowsum = jnp.sum(o2, axis=1)

    adj = jnp.zeros((r,), dtype=jnp.float32)
    for _ in range(FFT_TOPK):
        m = jnp.max(t2, axis=1, keepdims=True)
        cand = jnp.where(t2 == m, iota, fp)
        amin = jnp.min(cand, axis=1, keepdims=True)
        onehot = iota == amin
        osel = jnp.sum(jnp.where(onehot, o2, 0.0), axis=1)
        tsel = m[:, 0]
        adj = adj + tsel - 2.0 * jnp.sqrt(jnp.maximum(osel * tsel, 0.0))
        t2 = jnp.where(onehot, -1.0, t2)

    scale2 = 1.0 / float(n_valid)
    total = (rowsum + adj) * scale2
    rowloss = jnp.sqrt(jnp.maximum(total, 0.0))
    partial = jnp.sum(rowloss).reshape(1, 1)

    @pl.when(i == 0)
    def _init():
        out_ref[...] = jnp.zeros((1, 1), jnp.float32)

    out_ref[...] += partial


@functools.partial(jax.jit, static_argnames=("block_rows",))
def _fourier_loss(output, target, block_rows=256):
    b, n = output.shape
    f = n // 2 + 1
    fp = ((f + 127) // 128) * 128
    w = jnp.asarray(_dft_weights(n, fp), dtype=jnp.bfloat16)

    grid = (b // block_rows,)
    out = pl.pallas_call(
        functools.partial(_fourier_loss_block, f=f, fp=fp, n_valid=n),
        grid=grid,
        in_specs=[
            pl.BlockSpec((block_rows, n), lambda i: (i, 0)),
            pl.BlockSpec((block_rows, n), lambda i: (i, 0)),
            pl.BlockSpec((n, 2 * fp), lambda i: (0, 0)),
        ],
        out_specs=pl.BlockSpec((1, 1), lambda i: (0, 0)),
        out_shape=jax.ShapeDtypeStruct((1, 1), jnp.float32),
    )(output, target, w)
    return out[0, 0] / b


def kernel(output, target):
    return _fourier_loss(output, target)


# speedup vs baseline: 1.3682x; 1.1178x over previous
import functools
import math

import numpy as np
import jax
import jax.numpy as jnp
from jax.experimental import pallas as pl


FFT_TOPK = 8


def _dft_weights(n: int, fp: int) -> np.ndarray:
    f = n // 2 + 1
    kk = np.arange(f, dtype=np.float64)
    nn = np.arange(n, dtype=np.float64)
    ang = 2.0 * np.pi * np.outer(nn, kk) / n
    w = np.zeros((n, 2 * fp), dtype=np.float64)
    w[:, :f] = np.cos(ang)
    w[:, fp:fp + f] = np.sin(ang)
    return w.astype(np.float32)


def _fourier_loss_block(xo_ref, xt_ref, w_ref, out_ref, *, f, fp, n_valid):
    i = pl.program_id(0)

    w = w_ref[...]
    om = jnp.dot(xo_ref[...].astype(jnp.bfloat16), w,
                 preferred_element_type=jnp.float32)
    tm = jnp.dot(xt_ref[...].astype(jnp.bfloat16), w,
                 preferred_element_type=jnp.float32)
    del w

    o2 = om[:, :fp] ** 2 + om[:, fp:] ** 2
    t2 = tm[:, :fp] ** 2 + tm[:, fp:] ** 2

    r = o2.shape[0]
    iota = jax.lax.broadcasted_iota(jnp.int32, (r, fp), 1)
    valid = iota < f
    t2 = jnp.where(valid, t2, -1.0)

    rowsum = jnp.sum(o2, axis=1)
    oabs = jnp.sqrt(o2)

    adj = jnp.zeros((r,), dtype=jnp.float32)
    for _ in range(FFT_TOPK):
        m = jnp.max(t2, axis=1, keepdims=True)
        sel = t2 == m
        c = 2.0 * jnp.sqrt(jnp.maximum(m, 0.0))
        adj = adj + jnp.sum(jnp.where(sel, m - c * oabs, 0.0), axis=1)
        t2 = jnp.where(sel, -1.0, t2)

    scale2 = 1.0 / float(n_valid)
    total = (rowsum + adj) * scale2
    rowloss = jnp.sqrt(jnp.maximum(total, 0.0))
    partial = jnp.sum(rowloss).reshape(1, 1)

    @pl.when(i == 0)
    def _init():
        out_ref[...] = jnp.zeros((1, 1), jnp.float32)

    out_ref[...] += partial


@functools.partial(jax.jit, static_argnames=("block_rows",))
def _fourier_loss(output, target, block_rows=256):
    b, n = output.shape
    f = n // 2 + 1
    fp = ((f + 127) // 128) * 128
    w = jnp.asarray(_dft_weights(n, fp), dtype=jnp.bfloat16)

    grid = (b // block_rows,)
    out = pl.pallas_call(
        functools.partial(_fourier_loss_block, f=f, fp=fp, n_valid=n),
        grid=grid,
        in_specs=[
            pl.BlockSpec((block_rows, n), lambda i: (i, 0)),
            pl.BlockSpec((block_rows, n), lambda i: (i, 0)),
            pl.BlockSpec((n, 2 * fp), lambda i: (0, 0)),
        ],
        out_specs=pl.BlockSpec((1, 1), lambda i: (0, 0)),
        out_shape=jax.ShapeDtypeStruct((1, 1), jnp.float32),
    )(output, target, w)
    return out[0, 0] / b


def kernel(output, target):
    return _fourier_loss(output, target)


# speedup vs baseline: 1.3820x; 1.0101x over previous
import functools
import math

import numpy as np
import jax
import jax.numpy as jnp
from jax.experimental import pallas as pl


FFT_TOPK = 8


def _dft_weights(n: int, fp: int) -> np.ndarray:
    f = n // 2 + 1
    kk = np.arange(f, dtype=np.float64)
    nn = np.arange(n, dtype=np.float64)
    ang = 2.0 * np.pi * np.outer(nn, kk) / n
    w = np.zeros((n, 2 * fp), dtype=np.float64)
    w[:, :f] = np.cos(ang)
    w[:, fp:fp + f] = np.sin(ang)
    return w.astype(np.float32)


def _fourier_loss_block(xo_ref, xt_ref, w_ref, out_ref, *, f, fp, n_valid):
    i = pl.program_id(0)

    w = w_ref[...]
    om = jnp.dot(xo_ref[...].astype(jnp.bfloat16), w,
                 preferred_element_type=jnp.float32)
    tm = jnp.dot(xt_ref[...].astype(jnp.bfloat16), w,
                 preferred_element_type=jnp.float32)
    del w

    o2 = om[:, :fp] ** 2 + om[:, fp:] ** 2
    t2 = tm[:, :fp] ** 2 + tm[:, fp:] ** 2

    r = o2.shape[0]
    iota = jax.lax.broadcasted_iota(jnp.int32, (r, fp), 1)
    valid = iota < f
    t2 = jnp.where(valid, t2, -1.0)

    rowsum = jnp.sum(o2, axis=1)
    oabs = jnp.sqrt(o2)

    adj = jnp.zeros((r,), dtype=jnp.float32)
    for _ in range(FFT_TOPK):
        m = jnp.max(t2, axis=1, keepdims=True)
        sel = t2 == m
        c = 2.0 * jnp.sqrt(jnp.maximum(m, 0.0))
        adj = adj + jnp.sum(jnp.where(sel, m - c * oabs, 0.0), axis=1)
        t2 = jnp.where(sel, -1.0, t2)

    scale2 = 1.0 / float(n_valid)
    total = (rowsum + adj) * scale2
    rowloss = jnp.sqrt(jnp.maximum(total, 0.0))
    partial = jnp.sum(rowloss).reshape(1, 1)

    @pl.when(i == 0)
    def _init():
        out_ref[...] = jnp.zeros((1, 1), jnp.float32)

    out_ref[...] += partial


@functools.partial(jax.jit, static_argnames=("block_rows",))
def _fourier_loss(output, target, block_rows=512):
    b, n = output.shape
    f = n // 2 + 1
    fp = ((f + 127) // 128) * 128
    w = jnp.asarray(_dft_weights(n, fp), dtype=jnp.bfloat16)

    grid = (b // block_rows,)
    out = pl.pallas_call(
        functools.partial(_fourier_loss_block, f=f, fp=fp, n_valid=n),
        grid=grid,
        in_specs=[
            pl.BlockSpec((block_rows, n), lambda i: (i, 0)),
            pl.BlockSpec((block_rows, n), lambda i: (i, 0)),
            pl.BlockSpec((n, 2 * fp), lambda i: (0, 0)),
        ],
        out_specs=pl.BlockSpec((1, 1), lambda i: (0, 0)),
        out_shape=jax.ShapeDtypeStruct((1, 1), jnp.float32),
    )(output, target, w)
    return out[0, 0] / b


def kernel(output, target):
    return _fourier_loss(output, target)
